# baseline (speedup 1.0000x reference)
import functools

import jax
import jax.numpy as jnp
from jax import lax
from jax.experimental import pallas as pl
from jax.experimental.pallas import tpu as pltpu
from jax.experimental.pallas import tpu_sc as plsc

QTYPE_LEN = 4
HIDDEN = 256
EMB = QTYPE_LEN * HIDDEN
VOCA = 8192
N = 4096
D_IN = 768

BN = 1024
BK = 2048
BC = 1024
NI = N // BN
NJ = VOCA // BK


def _argmax_body(raw_ref, w_ref, b_ref, tab_ref, idx_out_ref,
                 pred_ref, vmax_ref, vidx_ref):
    j = pl.program_id(0)
    i = pl.program_id(1)
    rows = pl.ds(i * BN, BN)

    @pl.when(j == 0)
    def _():
        pred_ref[rows, :] = lax.dot_general(
            raw_ref[...], w_ref[...],
            dimension_numbers=(((1,), (0,)), ((), ())),
            preferred_element_type=jnp.float32) + b_ref[...]

    pred = pred_ref[rows, :]
    for c in range(BK // BC):
        scores = lax.dot_general(
            pred, tab_ref[pl.ds(c * BC, BC), :],
            dimension_numbers=(((1,), (1,)), ((), ())),
            preferred_element_type=jnp.float32)

        local_max = jnp.max(scores, axis=1, keepdims=True)
        cols = lax.broadcasted_iota(jnp.int32, (BN, BC), 1)
        local_idx = jnp.min(
            jnp.where(scores == local_max, cols, jnp.int32(VOCA)),
            axis=1, keepdims=True) + (j * BK + c * BC)

        if c == 0:
            @pl.when(j == 0)
            def _():
                vmax_ref[rows, :] = local_max
                vidx_ref[rows, :] = local_idx

            @pl.when(j > 0)
            def _():
                prev_max = vmax_ref[rows, :]
                prev_idx = vidx_ref[rows, :]
                better = local_max > prev_max
                vmax_ref[rows, :] = jnp.where(better, local_max, prev_max)
                vidx_ref[rows, :] = jnp.where(better, local_idx, prev_idx)
        else:
            prev_max = vmax_ref[rows, :]
            prev_idx = vidx_ref[rows, :]
            better = local_max > prev_max
            vmax_ref[rows, :] = jnp.where(better, local_max, prev_max)
            vidx_ref[rows, :] = jnp.where(better, local_idx, prev_idx)

    idx_out_ref[...] = vidx_ref[rows, :].reshape(1, BN, 1)


def _argmax_indices(raw_rep, W, b2, table):
    return pl.pallas_call(
        _argmax_body,
        grid=(NJ, NI),
        in_specs=[
            pl.BlockSpec((BN, D_IN), lambda j, i: (i, 0)),
            pl.BlockSpec((D_IN, EMB), lambda j, i: (0, 0)),
            pl.BlockSpec((1, EMB), lambda j, i: (0, 0)),
            pl.BlockSpec((BK, EMB), lambda j, i: (j, 0)),
        ],
        out_specs=pl.BlockSpec((1, BN, 1), lambda j, i: (i, 0, 0)),
        out_shape=jax.ShapeDtypeStruct((NI, BN, 1), jnp.int32),
        scratch_shapes=[
            pltpu.VMEM((N, EMB), jnp.float32),
            pltpu.VMEM((N, 1), jnp.float32),
            pltpu.VMEM((N, 1), jnp.int32),
        ],
        compiler_params=pltpu.CompilerParams(
            dimension_semantics=("arbitrary", "arbitrary")),
    )(raw_rep, W, b2, table)


_NC = 2
_NS = 16
_NW = _NC * _NS
_BPW = N // _NW
_CHUNK = 64


def _gather_body(table_hbm, idx_hbm, out_hbm, idx_v, rows_v, sem):
    c = lax.axis_index("c")
    s = lax.axis_index("s")
    wid = s * _NC + c
    base = wid * _BPW
    for t in range(_BPW // _CHUNK):
        off = base + t * _CHUNK
        pltpu.sync_copy(idx_hbm.at[pl.ds(off, _CHUNK)], idx_v)
        pltpu.async_copy(table_hbm.at[idx_v], rows_v, sem).wait()
        for q in range(QTYPE_LEN):
            pltpu.sync_copy(rows_v.at[:, pl.ds(q * HIDDEN, HIDDEN)],
                            out_hbm.at[pl.ds(off, _CHUNK), q])


def _gather_rows(table3, idx):
    mesh = plsc.VectorSubcoreMesh(core_axis_name="c", subcore_axis_name="s")
    k = functools.partial(
        pl.kernel, mesh=mesh,
        out_type=jax.ShapeDtypeStruct((N, QTYPE_LEN, HIDDEN), jnp.float32),
        scratch_types=[
            pltpu.VMEM((_CHUNK,), jnp.int32),
            pltpu.VMEM((_CHUNK, EMB), jnp.float32),
            pltpu.SemaphoreType.DMA,
        ],
    )(_gather_body)
    return k(table3, idx)


def kernel(raw_rep, W, b, table):
    idx3 = _argmax_indices(raw_rep, W, b.reshape(1, EMB), table)
    idx = idx3.reshape(N)
    return _gather_rows(table, idx)


# speedup vs baseline: 1.0157x; 1.0157x over previous
import functools

import jax
import jax.numpy as jnp
from jax import lax
from jax.experimental import pallas as pl
from jax.experimental.pallas import tpu as pltpu
from jax.experimental.pallas import tpu_sc as plsc

QTYPE_LEN = 4
HIDDEN = 256
EMB = QTYPE_LEN * HIDDEN
VOCA = 8192
N = 4096
D_IN = 768

BN = 1024
BK = 2048
BC = 1024
NI = N // BN
NJ = VOCA // BK


def _pred_body(raw_ref, w_ref, b_ref, pred_ref):
    pred_ref[...] = lax.dot_general(
        raw_ref[...], w_ref[...],
        dimension_numbers=(((1,), (0,)), ((), ())),
        preferred_element_type=jnp.float32) + b_ref[...]


def _pred(raw_rep, W, b2):
    return pl.pallas_call(
        _pred_body,
        grid=(NI,),
        in_specs=[
            pl.BlockSpec((BN, D_IN), lambda i: (i, 0)),
            pl.BlockSpec((D_IN, EMB), lambda i: (0, 0)),
            pl.BlockSpec((1, EMB), lambda i: (0, 0)),
        ],
        out_specs=pl.BlockSpec((BN, EMB), lambda i: (i, 0)),
        out_shape=jax.ShapeDtypeStruct((N, EMB), jnp.float32),
    )(raw_rep, W, b2)


def _argmax_body(pred_ref, tab_ref, idx_out_ref, vmax_ref, vidx_ref):
    j = pl.program_id(0)
    i = pl.program_id(1)
    rows = pl.ds(i * BN, BN)

    scores = lax.dot_general(
        pred_ref[...], tab_ref[...],
        dimension_numbers=(((1,), (1,)), ((), ())),
        preferred_element_type=jnp.float32)

    local_max = jnp.max(scores, axis=1, keepdims=True)
    cols = lax.broadcasted_iota(jnp.int32, (BN, BK), 1)
    local_idx = jnp.min(
        jnp.where(scores == local_max, cols, jnp.int32(VOCA)),
        axis=1, keepdims=True) + j * BK

    @pl.when(j == 0)
    def _():
        vmax_ref[rows, :] = local_max
        vidx_ref[rows, :] = local_idx

    @pl.when(j > 0)
    def _():
        prev_max = vmax_ref[rows, :]
        prev_idx = vidx_ref[rows, :]
        better = local_max > prev_max
        vmax_ref[rows, :] = jnp.where(better, local_max, prev_max)
        vidx_ref[rows, :] = jnp.where(better, local_idx, prev_idx)

    idx_out_ref[...] = vidx_ref[rows, :].reshape(1, BN, 1)


def _argmax_indices(pred, table):
    return pl.pallas_call(
        _argmax_body,
        grid=(NJ, NI),
        in_specs=[
            pl.BlockSpec((BN, EMB), lambda j, i: (i, 0)),
            pl.BlockSpec((BK, EMB), lambda j, i: (j, 0)),
        ],
        out_specs=pl.BlockSpec((1, BN, 1), lambda j, i: (i, 0, 0)),
        out_shape=jax.ShapeDtypeStruct((NI, BN, 1), jnp.int32),
        scratch_shapes=[
            pltpu.VMEM((N, 1), jnp.float32),
            pltpu.VMEM((N, 1), jnp.int32),
        ],
        compiler_params=pltpu.CompilerParams(
            dimension_semantics=("arbitrary", "arbitrary")),
    )(pred, table)


_NC = 2
_NS = 16
_NW = _NC * _NS
_BPW = N // _NW
_CHUNK = 64


def _gather_body(table_hbm, idx_hbm, out_hbm, idx_v, rows_v, sem):
    c = lax.axis_index("c")
    s = lax.axis_index("s")
    wid = s * _NC + c
    base = wid * _BPW
    for t in range(_BPW // _CHUNK):
        off = base + t * _CHUNK
        pltpu.sync_copy(idx_hbm.at[pl.ds(off, _CHUNK)], idx_v)
        pltpu.async_copy(table_hbm.at[idx_v], rows_v, sem).wait()
        for q in range(QTYPE_LEN):
            pltpu.sync_copy(rows_v.at[:, pl.ds(q * HIDDEN, HIDDEN)],
                            out_hbm.at[pl.ds(off, _CHUNK), q])


def _gather_rows(table3, idx):
    mesh = plsc.VectorSubcoreMesh(core_axis_name="c", subcore_axis_name="s")
    k = functools.partial(
        pl.kernel, mesh=mesh,
        out_type=jax.ShapeDtypeStruct((N, QTYPE_LEN, HIDDEN), jnp.float32),
        scratch_types=[
            pltpu.VMEM((_CHUNK,), jnp.int32),
            pltpu.VMEM((_CHUNK, EMB), jnp.float32),
            pltpu.SemaphoreType.DMA,
        ],
    )(_gather_body)
    return k(table3, idx)


def kernel(raw_rep, W, b, table):
    pred = _pred(raw_rep, W, b.reshape(1, EMB))
    idx3 = _argmax_indices(pred, table)
    idx = idx3.reshape(N)
    return _gather_rows(table, idx)


# speedup vs baseline: 1.0615x; 1.0451x over previous
import functools

import jax
import jax.numpy as jnp
from jax import lax
from jax.experimental import pallas as pl
from jax.experimental.pallas import tpu as pltpu
from jax.experimental.pallas import tpu_sc as plsc

QTYPE_LEN = 4
HIDDEN = 256
EMB = QTYPE_LEN * HIDDEN
VOCA = 8192
N = 4096
D_IN = 768

BN = 1024
BK = 2048
BC = 1024
NI = N // BN
NJ = VOCA // BK


def _argmax_body(raw_ref, w_ref, b_ref, tab_ref, idx_out_ref,
                 pred_ref, vmax_ref, vidx_ref):
    j = pl.program_id(0)
    i = pl.program_id(1)
    rows = pl.ds(i * BN, BN)

    @pl.when(j == 0)
    def _():
        pred_ref[rows, :] = (lax.dot_general(
            raw_ref[...], w_ref[...],
            dimension_numbers=(((1,), (0,)), ((), ())),
            preferred_element_type=jnp.float32) + b_ref[...]
        ).astype(jnp.bfloat16)

    scores = lax.dot_general(
        pred_ref[rows, :], tab_ref[...].astype(jnp.bfloat16),
        dimension_numbers=(((1,), (1,)), ((), ())),
        preferred_element_type=jnp.float32)

    local_max = jnp.max(scores, axis=1, keepdims=True)
    cols = lax.broadcasted_iota(jnp.int32, (BN, BK), 1)
    local_idx = jnp.min(
        jnp.where(scores == local_max, cols, jnp.int32(VOCA)),
        axis=1, keepdims=True) + j * BK

    @pl.when(j == 0)
    def _():
        vmax_ref[rows, :] = local_max
        vidx_ref[rows, :] = local_idx

    @pl.when(j > 0)
    def _():
        prev_max = vmax_ref[rows, :]
        prev_idx = vidx_ref[rows, :]
        better = local_max > prev_max
        vmax_ref[rows, :] = jnp.where(better, local_max, prev_max)
        vidx_ref[rows, :] = jnp.where(better, local_idx, prev_idx)

    idx_out_ref[...] = vidx_ref[rows, :].reshape(1, BN, 1)


def _argmax_indices(raw_rep, W, b2, table):
    return pl.pallas_call(
        _argmax_body,
        grid=(NJ, NI),
        in_specs=[
            pl.BlockSpec((BN, D_IN), lambda j, i: (i, 0)),
            pl.BlockSpec((D_IN, EMB), lambda j, i: (0, 0)),
            pl.BlockSpec((1, EMB), lambda j, i: (0, 0)),
            pl.BlockSpec((BK, EMB), lambda j, i: (j, 0)),
        ],
        out_specs=pl.BlockSpec((1, BN, 1), lambda j, i: (i, 0, 0)),
        out_shape=jax.ShapeDtypeStruct((NI, BN, 1), jnp.int32),
        scratch_shapes=[
            pltpu.VMEM((N, EMB), jnp.bfloat16),
            pltpu.VMEM((N, 1), jnp.float32),
            pltpu.VMEM((N, 1), jnp.int32),
        ],
        compiler_params=pltpu.CompilerParams(
            dimension_semantics=("arbitrary", "arbitrary")),
    )(raw_rep, W, b2, table)


_NC = 2
_NS = 16
_NW = _NC * _NS
_BPW = N // _NW
_CHUNK = 64


def _gather_body(table_hbm, idx_hbm, out_hbm, idx_v, rows_v, sem):
    c = lax.axis_index("c")
    s = lax.axis_index("s")
    wid = s * _NC + c
    base = wid * _BPW
    for t in range(_BPW // _CHUNK):
        off = base + t * _CHUNK
        pltpu.sync_copy(idx_hbm.at[pl.ds(off, _CHUNK)], idx_v)
        pltpu.async_copy(table_hbm.at[idx_v], rows_v, sem).wait()
        for q in range(QTYPE_LEN):
            pltpu.sync_copy(rows_v.at[:, pl.ds(q * HIDDEN, HIDDEN)],
                            out_hbm.at[pl.ds(off, _CHUNK), q])


def _gather_rows(table3, idx):
    mesh = plsc.VectorSubcoreMesh(core_axis_name="c", subcore_axis_name="s")
    k = functools.partial(
        pl.kernel, mesh=mesh,
        out_type=jax.ShapeDtypeStruct((N, QTYPE_LEN, HIDDEN), jnp.float32),
        scratch_types=[
            pltpu.VMEM((_CHUNK,), jnp.int32),
            pltpu.VMEM((_CHUNK, EMB), jnp.float32),
            pltpu.SemaphoreType.DMA,
        ],
    )(_gather_body)
    return k(table3, idx)


def kernel(raw_rep, W, b, table):
    idx3 = _argmax_indices(raw_rep, W, b.reshape(1, EMB), table)
    idx = idx3.reshape(N)
    return _gather_rows(table, idx)
